# baseline (speedup 1.0000x reference)
import functools

import numpy as np
import jax
import jax.numpy as jnp
from jax import lax
from jax.experimental import pallas as pl
from jax.experimental.pallas import tpu as pltpu
from jax.experimental.pallas import tpu_sc as plsc

_F32 = jnp.float32
_BF16 = jnp.bfloat16
_LATENT = 64
_K = 1024
_BN_EPS = 1e-3


def _s2d(x):
    B, H, W, C = x.shape
    x = x.reshape(B, H // 2, 2, W // 2, 2, C)
    x = x.transpose(0, 1, 3, 2, 4, 5)
    return x.reshape(B, H // 2, W // 2, 4 * C)


def _make_wg(w):
    Cin, Cout = w.shape[2], w.shape[3]
    mats = []
    for gi in range(2):
        for gj in range(2):
            rows = []
            for pi in range(2):
                for pj in range(2):
                    di, dj = 2 * gi + pi, 2 * gj + pj
                    if di <= 2 and dj <= 2:
                        rows.append(w[di, dj])
                    else:
                        rows.append(jnp.zeros((Cin, Cout), w.dtype))
            mats.append(jnp.concatenate(rows, axis=0))
    return jnp.stack(mats).astype(_BF16)


def _enc_body(xp_ref, wg_ref, b_ref, o_ref):
    H, W, Cout = o_ref.shape[1], o_ref.shape[2], o_ref.shape[3]
    C4 = xp_ref.shape[3]
    acc = jnp.zeros((H * W, Cout), _F32)
    for g in range(4):
        gi, gj = g // 2, g % 2
        xs = xp_ref[0, gi:gi + H, gj:gj + W, :].reshape(H * W, C4)
        acc = acc + jnp.dot(xs, wg_ref[g], preferred_element_type=_F32)
    acc = acc + b_ref[0]
    o_ref[0] = jnp.maximum(acc, 0.0).astype(_BF16).reshape(H, W, Cout)


def _enc_conv(xgp, wg, b, H, W, Cout):
    B = xgp.shape[0]
    C4 = xgp.shape[3]
    return pl.pallas_call(
        _enc_body,
        grid=(B,),
        in_specs=[
            pl.BlockSpec((1, H + 1, W + 1, C4), lambda n: (n, 0, 0, 0)),
            pl.BlockSpec((4, C4, Cout), lambda n: (0, 0, 0)),
            pl.BlockSpec((1, Cout), lambda n: (0, 0)),
        ],
        out_specs=pl.BlockSpec((1, H, W, Cout), lambda n: (n, 0, 0, 0)),
        out_shape=jax.ShapeDtypeStruct((B, H, W, Cout), _BF16),
    )(xgp, wg, b.reshape(1, Cout))


def _vq_body(xp_ref, wg_ref, b2_ref, w3_ref, b3_ref, emb_ref, idx_ref):
    C4 = xp_ref.shape[3]
    acc = jnp.zeros((3136, _LATENT), _F32)
    for g in range(4):
        gi, gj = g // 2, g % 2
        xs = xp_ref[0, gi:gi + 56, gj:gj + 56, :].reshape(3136, C4)
        acc = acc + jnp.dot(xs, wg_ref[g], preferred_element_type=_F32)
    h = jnp.maximum(acc + b2_ref[0], 0.0).astype(_BF16)
    z = jnp.dot(h, w3_ref[...], preferred_element_type=_F32) + b3_ref[0]
    e = emb_ref[...]
    sim = jnp.dot(z.astype(_BF16), e, preferred_element_type=_F32)
    ef = e.astype(_F32)
    e2 = jnp.sum(ef * ef, axis=0)
    dist = e2[None, :] - 2.0 * sim
    m = jnp.min(dist, axis=1, keepdims=True)
    iota = lax.broadcasted_iota(jnp.int32, dist.shape, 1)
    idx = jnp.min(jnp.where(dist <= m, iota, jnp.int32(_K)), axis=1)
    idx_ref[0] = idx.reshape(8, 392)


def _vq_quantize(h1g, wg2, b2, w3, b3, emb):
    B = h1g.shape[0]
    C4 = h1g.shape[3]
    return pl.pallas_call(
        _vq_body,
        grid=(B,),
        in_specs=[
            pl.BlockSpec((1, 57, 57, C4), lambda n: (n, 0, 0, 0)),
            pl.BlockSpec((4, C4, _LATENT), lambda n: (0, 0, 0)),
            pl.BlockSpec((1, _LATENT), lambda n: (0, 0)),
            pl.BlockSpec((_LATENT, _LATENT), lambda n: (0, 0)),
            pl.BlockSpec((1, _LATENT), lambda n: (0, 0)),
            pl.BlockSpec((_LATENT, _K), lambda n: (0, 0)),
        ],
        out_specs=pl.BlockSpec((1, 8, 392), lambda n: (n, 0, 0)),
        out_shape=jax.ShapeDtypeStruct((B, 8, 392), jnp.int32),
    )(h1g, wg2, b2.reshape(1, _LATENT), w3.astype(_BF16),
      b3.reshape(1, _LATENT), emb.astype(_BF16))


def _sc_gather(table, idx):
    B, D = idx.shape[0], table.shape[1]
    DP = 128
    table_p = jnp.pad(table, ((0, 0), (0, DP - D)))
    info = plsc.get_sparse_core_info()
    NC, NS = info.num_cores, info.num_subcores
    NW = NC * NS
    b_per_w = B // NW
    mesh = plsc.VectorSubcoreMesh(core_axis_name="c", subcore_axis_name="s")

    @functools.partial(
        pl.kernel, mesh=mesh,
        out_type=jax.ShapeDtypeStruct((B, DP), _F32),
        scratch_types=[
            pltpu.VMEM((b_per_w,), jnp.int32),
            pltpu.VMEM((b_per_w, DP), _F32),
            pltpu.VMEM_SHARED((_K, DP), _F32),
            pltpu.SemaphoreType.DMA,
        ],
    )
    def k(table_hbm, idx_hbm, out_hbm, idx_v, rows_v, tab_s, sem):
        wid = lax.axis_index("s") * NC + lax.axis_index("c")
        base = wid * b_per_w
        @pl.when(lax.axis_index("s") == 0)
        def _():
            pltpu.sync_copy(table_hbm, tab_s)
        pltpu.sync_copy(idx_hbm.at[pl.ds(base, b_per_w)], idx_v)
        plsc.subcore_barrier()
        ch = 112
        cps = [
            pltpu.async_copy(tab_s.at[idx_v.at[pl.ds(k * ch, ch)]],
                             rows_v.at[pl.ds(k * ch, ch)], sem)
            for k in range(b_per_w // ch)
        ]
        for cp in cps:
            cp.wait()
        pltpu.sync_copy(rows_v, out_hbm.at[pl.ds(base, b_per_w)])

    return k(table_p, idx)[:, :D]


def _dect_body(xp_ref, w_ref, b_ref, o_ref):
    H, W, Cout = o_ref.shape[3], o_ref.shape[4], o_ref.shape[5]
    Cin = xp_ref.shape[3]
    xm = xp_ref[0, 0:H, 0:W, :].reshape(H * W, Cin)
    xm0 = xp_ref[0, 0:H, 1:W + 1, :].reshape(H * W, Cin)
    x0m = xp_ref[0, 1:H + 1, 0:W, :].reshape(H * W, Cin)
    x00 = xp_ref[0, 1:H + 1, 1:W + 1, :].reshape(H * W, Cin)
    b = b_ref[0]

    def mm(a, wa):
        return jnp.dot(a, wa, preferred_element_type=_F32)

    def act(v):
        v = v + b
        return jnp.where(v >= 0, v, 0.3 * v).astype(_BF16).reshape(H, W, Cout)

    o_ref[0, 0, 0] = act(mm(xm, w_ref[0, 0]) + mm(xm0, w_ref[0, 2])
                         + mm(x0m, w_ref[2, 0]) + mm(x00, w_ref[2, 2]))
    o_ref[0, 0, 1] = act(mm(xm0, w_ref[0, 1]) + mm(x00, w_ref[2, 1]))
    o_ref[0, 1, 0] = act(mm(x0m, w_ref[1, 0]) + mm(x00, w_ref[1, 2]))
    o_ref[0, 1, 1] = act(mm(x00, w_ref[1, 1]))


def _dect_conv(xp, w, b, H, W, Cout):
    B = xp.shape[0]
    Cin = xp.shape[3]
    return pl.pallas_call(
        _dect_body,
        grid=(B,),
        in_specs=[
            pl.BlockSpec((1, H + 1, W + 1, Cin), lambda n: (n, 0, 0, 0)),
            pl.BlockSpec((3, 3, Cin, Cout), lambda n: (0, 0, 0, 0)),
            pl.BlockSpec((1, Cout), lambda n: (0, 0)),
        ],
        out_specs=pl.BlockSpec((1, 2, 2, H, W, Cout), lambda n: (n, 0, 0, 0, 0, 0)),
        out_shape=jax.ShapeDtypeStruct((B, 2, 2, H, W, Cout), _BF16),
    )(xp, w.astype(_BF16), b.reshape(1, Cout))


_T2 = {0: [(0, -1), (2, 0)], 1: [(1, 0)]}
_WIN2 = ((1, 0), (0, 1), (1, 1))


def _dec2_body(xp_ref, w_ref, b_ref, o_ref):
    Cin = xp_ref.shape[5]
    H = o_ref.shape[3]
    HW = H * H
    b = b_ref[0]
    wins = {}
    for (pu, su) in _WIN2:
        for (pv, sv) in _WIN2:
            wins[(pu, su, pv, sv)] = (
                xp_ref[0, pu, pv, su:su + H, sv:sv + H, :].reshape(HW, Cin))
    for e in range(2):
        for f in range(2):
            planes = []
            for ap in range(2):
                for bp in range(2):
                    acc = jnp.zeros((HW, 64), _F32)
                    for (r, du) in _T2[e]:
                        t = ap + du
                        pu, su = t % 2, (t - t % 2) // 2 + 1
                        for (c, dv) in _T2[f]:
                            s = bp + dv
                            pv, sv = s % 2, (s - s % 2) // 2 + 1
                            acc = acc + jnp.dot(wins[(pu, su, pv, sv)],
                                                w_ref[r, c],
                                                preferred_element_type=_F32)
                    v = acc + b
                    planes.append(jnp.where(v >= 0, v, 0.3 * v).astype(_BF16))
            o_ref[0, e, f] = jnp.concatenate(planes, axis=1).reshape(H, H, 256)


def _dec2_conv(xp, w, b):
    B = xp.shape[0]
    Cin = xp.shape[5]
    return pl.pallas_call(
        _dec2_body,
        grid=(B,),
        in_specs=[
            pl.BlockSpec((1, 2, 2, 57, 57, Cin), lambda n: (n, 0, 0, 0, 0, 0)),
            pl.BlockSpec((3, 3, Cin, 64), lambda n: (0, 0, 0, 0)),
            pl.BlockSpec((1, 64), lambda n: (0, 0)),
        ],
        out_specs=pl.BlockSpec((1, 2, 2, 56, 56, 256), lambda n: (n, 0, 0, 0, 0, 0)),
        out_shape=jax.ShapeDtypeStruct((B, 2, 2, 56, 56, 256), _BF16),
    )(xp, w.astype(_BF16), b.reshape(1, 64))


def _d3_axis_pairs():
    m = {}
    for g in (0, 1):
        for dlt in (-1, 0, 1):
            e = (g + dlt) % 2
            for ap in (0, 1):
                t = ap + (g + dlt - e) // 2
                a2 = t % 2
                su = (t - a2) // 2 + 1
                m.setdefault((e, su), []).append((g, ap, dlt, a2))
    return m


_D3_PAIRS = _d3_axis_pairs()


def _make_w3(w3):
    S = np.zeros((2, 2, 3, 3, 4, 16, 3, 3), np.float32)
    for (e, su), rows in _D3_PAIRS.items():
        for (f, sv), cols in _D3_PAIRS.items():
            for (g, ap, dlt, a2) in rows:
                for (h, bp, eps, b2) in cols:
                    col = g * 8 + h * 4 + ap * 2 + bp
                    S[e, f, su, sv, a2 * 2 + b2, col, dlt + 1, eps + 1] = 1.0
    W = jnp.einsum('efuvkpab,abc->efuvkcp', jnp.asarray(S), w3[:, :, :, 0])
    return W.reshape(2, 2, 3, 3, 256, 16).astype(_BF16)


def _dec3_body(xp_ref, w_ref, b_ref, o_ref):
    H = 56
    HW = H * H
    acc = jnp.zeros((HW, 16), _F32)
    for (e, su) in _D3_PAIRS:
        for (f, sv) in _D3_PAIRS:
            win = xp_ref[0, e, f, su:su + H, sv:sv + H, :].reshape(HW, 256)
            acc = acc + jnp.dot(win, w_ref[e, f, su, sv],
                                preferred_element_type=_F32)
    v = acc + b_ref[0]
    o_ref[0] = jax.nn.sigmoid(v).reshape(H, H, 16)


def _dec3_conv(xp, w16, b):
    B = xp.shape[0]
    return pl.pallas_call(
        _dec3_body,
        grid=(B,),
        in_specs=[
            pl.BlockSpec((1, 2, 2, 58, 58, 256), lambda n: (n, 0, 0, 0, 0, 0)),
            pl.BlockSpec((2, 2, 3, 3, 256, 16), lambda n: (0, 0, 0, 0, 0, 0)),
            pl.BlockSpec((1, 1), lambda n: (0, 0)),
        ],
        out_specs=pl.BlockSpec((1, 56, 56, 16), lambda n: (n, 0, 0, 0)),
        out_shape=jax.ShapeDtypeStruct((B, 56, 56, 16), _F32),
    )(xp, w16, b.reshape(1, 1))


def kernel(x, params):
    p = params
    B = x.shape[0]

    wg1 = _make_wg(p['enc_w1'])
    wg2 = _make_wg(p['enc_w2'])
    w3 = p['enc_w3'].reshape(_LATENT, _LATENT)
    bn_s1 = p['bn1_g'] * (1.0 / jnp.sqrt(1.0 + _BN_EPS))
    w_d1 = p['dec_w1'] * bn_s1
    b_d1 = p['dec_b1'] * bn_s1 + p['bn1_b']
    bn_s2 = p['bn2_g'] * (1.0 / jnp.sqrt(1.0 + _BN_EPS))
    w_d2 = p['dec_w2'] * bn_s2
    b_d2 = p['dec_b2'] * bn_s2 + p['bn2_b']
    w_d3 = _make_w3(p['dec_w3'])

    xg = _s2d(x.astype(_BF16))
    xgp = jnp.pad(xg, ((0, 0), (0, 1), (0, 1), (0, 0)))
    h1 = _enc_conv(xgp, wg1, p['enc_b1'], 112, 112, 32)
    h1g = jnp.pad(_s2d(h1), ((0, 0), (0, 1), (0, 1), (0, 0)))

    idx = _vq_quantize(h1g, wg2, p['enc_b2'], w3, p['enc_b3'], p['emb'])
    q = _sc_gather(p['emb'].T, idx.reshape(B * 3136))
    q = q.astype(_BF16).reshape(B, 56, 56, _LATENT)

    qp = jnp.pad(q, ((0, 0), (1, 0), (1, 0), (0, 0)))
    d1 = _dect_conv(qp, w_d1, b_d1, 56, 56, 128)
    d1p = jnp.pad(d1, ((0, 0), (0, 0), (0, 0), (1, 0), (1, 0), (0, 0)))
    d2 = _dec2_conv(d1p, w_d2, b_d2)
    d2p = jnp.pad(d2, ((0, 0), (0, 0), (0, 0), (1, 1), (1, 1), (0, 0)))
    o = _dec3_conv(d2p, w_d3, p['dec_b3'])
    o = o.reshape(B, 56, 56, 2, 2, 2, 2)
    out = o.transpose(0, 1, 5, 3, 2, 6, 4).reshape(B, 224, 224, 1)
    return out


# speedup vs baseline: 1.0161x; 1.0161x over previous
import functools

import numpy as np
import jax
import jax.numpy as jnp
from jax import lax
from jax.experimental import pallas as pl
from jax.experimental.pallas import tpu as pltpu
from jax.experimental.pallas import tpu_sc as plsc

_F32 = jnp.float32
_BF16 = jnp.bfloat16
_LATENT = 64
_K = 1024
_BN_EPS = 1e-3


def _s2d(x):
    B, H, W, C = x.shape
    x = x.reshape(B, H // 2, 2, W // 2, 2, C)
    x = x.transpose(0, 1, 3, 2, 4, 5)
    return x.reshape(B, H // 2, W // 2, 4 * C)


def _make_wg(w):
    Cin, Cout = w.shape[2], w.shape[3]
    mats = []
    for gi in range(2):
        for gj in range(2):
            rows = []
            for pi in range(2):
                for pj in range(2):
                    di, dj = 2 * gi + pi, 2 * gj + pj
                    if di <= 2 and dj <= 2:
                        rows.append(w[di, dj])
                    else:
                        rows.append(jnp.zeros((Cin, Cout), w.dtype))
            mats.append(jnp.concatenate(rows, axis=0))
    return jnp.stack(mats).astype(_BF16)


def _enc_body(xp_ref, wg_ref, b_ref, o_ref):
    H, W, Cout = o_ref.shape[1], o_ref.shape[2], o_ref.shape[3]
    C4 = xp_ref.shape[3]
    acc = jnp.zeros((H * W, Cout), _F32)
    for g in range(4):
        gi, gj = g // 2, g % 2
        xs = xp_ref[0, gi:gi + H, gj:gj + W, :].reshape(H * W, C4)
        acc = acc + jnp.dot(xs, wg_ref[g], preferred_element_type=_F32)
    acc = acc + b_ref[0]
    o_ref[0] = jnp.maximum(acc, 0.0).astype(_BF16).reshape(H, W, Cout)


def _enc_conv(xgp, wg, b, H, W, Cout):
    B = xgp.shape[0]
    C4 = xgp.shape[3]
    return pl.pallas_call(
        _enc_body,
        grid=(B,),
        in_specs=[
            pl.BlockSpec((1, H + 1, W + 1, C4), lambda n: (n, 0, 0, 0)),
            pl.BlockSpec((4, C4, Cout), lambda n: (0, 0, 0)),
            pl.BlockSpec((1, Cout), lambda n: (0, 0)),
        ],
        out_specs=pl.BlockSpec((1, H, W, Cout), lambda n: (n, 0, 0, 0)),
        out_shape=jax.ShapeDtypeStruct((B, H, W, Cout), _BF16),
    )(xgp, wg, b.reshape(1, Cout))


def _vq_body(h_ref, w3_ref, b3_ref, emb_ref, idx_ref):
    h = h_ref[0]
    z = jnp.dot(h, w3_ref[...], preferred_element_type=_F32) + b3_ref[0]
    e = emb_ref[...]
    sim = jnp.dot(z.astype(_BF16), e, preferred_element_type=_F32)
    ef = e.astype(_F32)
    e2 = jnp.sum(ef * ef, axis=0)
    dist = e2[None, :] - 2.0 * sim
    m = jnp.min(dist, axis=1, keepdims=True)
    iota = lax.broadcasted_iota(jnp.int32, dist.shape, 1)
    idx = jnp.min(jnp.where(dist <= m, iota, jnp.int32(_K)), axis=1)
    idx_ref[0] = idx.reshape(8, 392)


def _vq_quantize(h2, w3, b3, emb):
    B = h2.shape[0]
    return pl.pallas_call(
        _vq_body,
        grid=(B,),
        in_specs=[
            pl.BlockSpec((1, 3136, _LATENT), lambda n: (n, 0, 0)),
            pl.BlockSpec((_LATENT, _LATENT), lambda n: (0, 0)),
            pl.BlockSpec((1, _LATENT), lambda n: (0, 0)),
            pl.BlockSpec((_LATENT, _K), lambda n: (0, 0)),
        ],
        out_specs=pl.BlockSpec((1, 8, 392), lambda n: (n, 0, 0)),
        out_shape=jax.ShapeDtypeStruct((B, 8, 392), jnp.int32),
    )(h2, w3.astype(_BF16), b3.reshape(1, _LATENT), emb.astype(_BF16))


def _sc_gather(table, idx):
    B, D = idx.shape[0], table.shape[1]
    DP = 128
    table_p = jnp.pad(table, ((0, 0), (0, DP - D)))
    info = plsc.get_sparse_core_info()
    NC, NS = info.num_cores, info.num_subcores
    NW = NC * NS
    b_per_w = B // NW
    mesh = plsc.VectorSubcoreMesh(core_axis_name="c", subcore_axis_name="s")

    @functools.partial(
        pl.kernel, mesh=mesh,
        out_type=jax.ShapeDtypeStruct((B, DP), _F32),
        scratch_types=[
            pltpu.VMEM((b_per_w,), jnp.int32),
            pltpu.VMEM((b_per_w, DP), _F32),
            pltpu.VMEM_SHARED((_K, DP), _F32),
            pltpu.SemaphoreType.DMA,
        ],
    )
    def k(table_hbm, idx_hbm, out_hbm, idx_v, rows_v, tab_s, sem):
        wid = lax.axis_index("s") * NC + lax.axis_index("c")
        base = wid * b_per_w
        @pl.when(lax.axis_index("s") == 0)
        def _():
            pltpu.sync_copy(table_hbm, tab_s)
        pltpu.sync_copy(idx_hbm.at[pl.ds(base, b_per_w)], idx_v)
        plsc.subcore_barrier()
        ch = 112
        cps = [
            pltpu.async_copy(tab_s.at[idx_v.at[pl.ds(k * ch, ch)]],
                             rows_v.at[pl.ds(k * ch, ch)], sem)
            for k in range(b_per_w // ch)
        ]
        for cp in cps:
            cp.wait()
        pltpu.sync_copy(rows_v, out_hbm.at[pl.ds(base, b_per_w)])

    return k(table_p, idx)[:, :D]


def _dect_body(xp_ref, w_ref, b_ref, o_ref):
    H, W, Cout = o_ref.shape[3], o_ref.shape[4], o_ref.shape[5]
    Cin = xp_ref.shape[3]
    xm = xp_ref[0, 0:H, 0:W, :].reshape(H * W, Cin)
    xm0 = xp_ref[0, 0:H, 1:W + 1, :].reshape(H * W, Cin)
    x0m = xp_ref[0, 1:H + 1, 0:W, :].reshape(H * W, Cin)
    x00 = xp_ref[0, 1:H + 1, 1:W + 1, :].reshape(H * W, Cin)
    b = b_ref[0]

    def mm(a, wa):
        return jnp.dot(a, wa, preferred_element_type=_F32)

    def act(v):
        v = v + b
        return jnp.where(v >= 0, v, 0.3 * v).astype(_BF16).reshape(H, W, Cout)

    o_ref[0, 0, 0] = act(mm(xm, w_ref[0, 0]) + mm(xm0, w_ref[0, 2])
                         + mm(x0m, w_ref[2, 0]) + mm(x00, w_ref[2, 2]))
    o_ref[0, 0, 1] = act(mm(xm0, w_ref[0, 1]) + mm(x00, w_ref[2, 1]))
    o_ref[0, 1, 0] = act(mm(x0m, w_ref[1, 0]) + mm(x00, w_ref[1, 2]))
    o_ref[0, 1, 1] = act(mm(x00, w_ref[1, 1]))


def _dect_conv(xp, w, b, H, W, Cout):
    B = xp.shape[0]
    Cin = xp.shape[3]
    return pl.pallas_call(
        _dect_body,
        grid=(B,),
        in_specs=[
            pl.BlockSpec((1, H + 1, W + 1, Cin), lambda n: (n, 0, 0, 0)),
            pl.BlockSpec((3, 3, Cin, Cout), lambda n: (0, 0, 0, 0)),
            pl.BlockSpec((1, Cout), lambda n: (0, 0)),
        ],
        out_specs=pl.BlockSpec((1, 2, 2, H, W, Cout), lambda n: (n, 0, 0, 0, 0, 0)),
        out_shape=jax.ShapeDtypeStruct((B, 2, 2, H, W, Cout), _BF16),
    )(xp, w.astype(_BF16), b.reshape(1, Cout))


_T2 = {0: [(0, -1), (2, 0)], 1: [(1, 0)]}
_WIN2 = ((1, 0), (0, 1), (1, 1))


def _dec2_body(xp_ref, w_ref, b_ref, o_ref):
    Cin = xp_ref.shape[5]
    H = o_ref.shape[3]
    HW = H * H
    b = b_ref[0]
    wins = {}
    for (pu, su) in _WIN2:
        for (pv, sv) in _WIN2:
            wins[(pu, su, pv, sv)] = (
                xp_ref[0, pu, pv, su:su + H, sv:sv + H, :].reshape(HW, Cin))
    for e in range(2):
        for f in range(2):
            planes = []
            for ap in range(2):
                for bp in range(2):
                    acc = jnp.zeros((HW, 64), _F32)
                    for (r, du) in _T2[e]:
                        t = ap + du
                        pu, su = t % 2, (t - t % 2) // 2 + 1
                        for (c, dv) in _T2[f]:
                            s = bp + dv
                            pv, sv = s % 2, (s - s % 2) // 2 + 1
                            acc = acc + jnp.dot(wins[(pu, su, pv, sv)],
                                                w_ref[r, c],
                                                preferred_element_type=_F32)
                    v = acc + b
                    planes.append(jnp.where(v >= 0, v, 0.3 * v).astype(_BF16))
            o_ref[0, e, f] = jnp.concatenate(planes, axis=1).reshape(H, H, 256)


def _dec2_conv(xp, w, b):
    B = xp.shape[0]
    Cin = xp.shape[5]
    return pl.pallas_call(
        _dec2_body,
        grid=(B,),
        in_specs=[
            pl.BlockSpec((1, 2, 2, 57, 57, Cin), lambda n: (n, 0, 0, 0, 0, 0)),
            pl.BlockSpec((3, 3, Cin, 64), lambda n: (0, 0, 0, 0)),
            pl.BlockSpec((1, 64), lambda n: (0, 0)),
        ],
        out_specs=pl.BlockSpec((1, 2, 2, 56, 56, 256), lambda n: (n, 0, 0, 0, 0, 0)),
        out_shape=jax.ShapeDtypeStruct((B, 2, 2, 56, 56, 256), _BF16),
    )(xp, w.astype(_BF16), b.reshape(1, 64))


def _d3_axis_pairs():
    m = {}
    for g in (0, 1):
        for dlt in (-1, 0, 1):
            e = (g + dlt) % 2
            for ap in (0, 1):
                t = ap + (g + dlt - e) // 2
                a2 = t % 2
                su = (t - a2) // 2 + 1
                m.setdefault((e, su), []).append((g, ap, dlt, a2))
    return m


_D3_PAIRS = _d3_axis_pairs()


def _make_w3(w3):
    S = np.zeros((2, 2, 3, 3, 4, 16, 3, 3), np.float32)
    for (e, su), rows in _D3_PAIRS.items():
        for (f, sv), cols in _D3_PAIRS.items():
            for (g, ap, dlt, a2) in rows:
                for (h, bp, eps, b2) in cols:
                    col = g * 8 + h * 4 + ap * 2 + bp
                    S[e, f, su, sv, a2 * 2 + b2, col, dlt + 1, eps + 1] = 1.0
    W = jnp.einsum('efuvkpab,abc->efuvkcp', jnp.asarray(S), w3[:, :, :, 0])
    return W.reshape(2, 2, 3, 3, 256, 16).astype(_BF16)


def _dec3_body(xp_ref, w_ref, b_ref, o_ref):
    H = 56
    HW = H * H
    acc = jnp.zeros((HW, 16), _F32)
    for (e, su) in _D3_PAIRS:
        for (f, sv) in _D3_PAIRS:
            win = xp_ref[0, e, f, su:su + H, sv:sv + H, :].reshape(HW, 256)
            acc = acc + jnp.dot(win, w_ref[e, f, su, sv],
                                preferred_element_type=_F32)
    v = acc + b_ref[0]
    o_ref[0] = jax.nn.sigmoid(v).reshape(H, H, 16)


def _dec3_conv(xp, w16, b):
    B = xp.shape[0]
    return pl.pallas_call(
        _dec3_body,
        grid=(B,),
        in_specs=[
            pl.BlockSpec((1, 2, 2, 58, 58, 256), lambda n: (n, 0, 0, 0, 0, 0)),
            pl.BlockSpec((2, 2, 3, 3, 256, 16), lambda n: (0, 0, 0, 0, 0, 0)),
            pl.BlockSpec((1, 1), lambda n: (0, 0)),
        ],
        out_specs=pl.BlockSpec((1, 56, 56, 16), lambda n: (n, 0, 0, 0)),
        out_shape=jax.ShapeDtypeStruct((B, 56, 56, 16), _F32),
    )(xp, w16, b.reshape(1, 1))


def kernel(x, params):
    p = params
    B = x.shape[0]

    wg1 = _make_wg(p['enc_w1'])
    wg2 = _make_wg(p['enc_w2'])
    w3 = p['enc_w3'].reshape(_LATENT, _LATENT)
    bn_s1 = p['bn1_g'] * (1.0 / jnp.sqrt(1.0 + _BN_EPS))
    w_d1 = p['dec_w1'] * bn_s1
    b_d1 = p['dec_b1'] * bn_s1 + p['bn1_b']
    bn_s2 = p['bn2_g'] * (1.0 / jnp.sqrt(1.0 + _BN_EPS))
    w_d2 = p['dec_w2'] * bn_s2
    b_d2 = p['dec_b2'] * bn_s2 + p['bn2_b']
    w_d3 = _make_w3(p['dec_w3'])

    xg = _s2d(x.astype(_BF16))
    xgp = jnp.pad(xg, ((0, 0), (0, 1), (0, 1), (0, 0)))
    h1 = _enc_conv(xgp, wg1, p['enc_b1'], 112, 112, 32)
    h1g = jnp.pad(_s2d(h1), ((0, 0), (0, 1), (0, 1), (0, 0)))
    h2 = _enc_conv(h1g, wg2, p['enc_b2'], 56, 56, 64)

    idx = _vq_quantize(h2.reshape(B, 3136, _LATENT), w3, p['enc_b3'], p['emb'])
    q = _sc_gather(p['emb'].T, idx.reshape(B * 3136))
    q = q.astype(_BF16).reshape(B, 56, 56, _LATENT)

    qp = jnp.pad(q, ((0, 0), (1, 0), (1, 0), (0, 0)))
    d1 = _dect_conv(qp, w_d1, b_d1, 56, 56, 128)
    d1p = jnp.pad(d1, ((0, 0), (0, 0), (0, 0), (1, 0), (1, 0), (0, 0)))
    d2 = _dec2_conv(d1p, w_d2, b_d2)
    d2p = jnp.pad(d2, ((0, 0), (0, 0), (0, 0), (1, 1), (1, 1), (0, 0)))
    o = _dec3_conv(d2p, w_d3, p['dec_b3'])
    o = o.reshape(B, 56, 56, 2, 2, 2, 2)
    out = o.transpose(0, 1, 5, 3, 2, 6, 4).reshape(B, 224, 224, 1)
    return out
